# Initial kernel scaffold; baseline (speedup 1.0000x reference)
import functools

import jax
import jax.numpy as jnp
from jax import lax
from jax.experimental import pallas as pl
from jax.experimental.pallas import tpu as pltpu
from jax.experimental.pallas import tpu_sc as plsc

N = 10000
D = 128
E = 320000

NC = 2
NS = 16
NW = NC * NS
CHUNK = 80
NROWS = E // CHUNK
RPT = NROWS // NW
ROW_SLICE = N // NS
NP_DEG = 10240
DEG_SLICE = NP_DEG // NS

_MESH = plsc.VectorSubcoreMesh(core_axis_name="c", subcore_axis_name="s")


@functools.partial(
    pl.kernel,
    out_type=jax.ShapeDtypeStruct((NC, NP_DEG), jnp.float32),
    mesh=_MESH,
    scratch_types=[
        pltpu.VMEM((RPT, CHUNK), jnp.int32),
        pltpu.VMEM((CHUNK,), jnp.float32),
        pltpu.VMEM((DEG_SLICE,), jnp.float32),
        pltpu.VMEM_SHARED((NP_DEG,), jnp.float32),
    ],
)
def _deg_kernel(dst_hbm, out_hbm, dst_v, ones_v, zero_v, acc_sh):
    c = lax.axis_index("c")
    s = lax.axis_index("s")
    wid = c * NS + s

    for j in range(CHUNK // 16):
        ones_v[pl.ds(j * 16, 16)] = jnp.full((16,), 1.0, jnp.float32)

    def zfill(k, carry):
        zero_v[pl.ds(k * 16, 16)] = jnp.zeros((16,), jnp.float32)
        return carry

    lax.fori_loop(0, DEG_SLICE // 16, zfill, 0)
    pltpu.sync_copy(zero_v, acc_sh.at[pl.ds(s * DEG_SLICE, DEG_SLICE)])
    pltpu.sync_copy(dst_hbm.at[pl.ds(wid * RPT, RPT)], dst_v)
    plsc.subcore_barrier()

    def body(j, carry):
        pltpu.sync_copy(ones_v, acc_sh.at[dst_v.at[j]], add=True)
        return carry

    lax.fori_loop(0, RPT, body, 0)
    plsc.subcore_barrier()
    pltpu.sync_copy(
        acc_sh.at[pl.ds(s * DEG_SLICE, DEG_SLICE)],
        out_hbm.at[c, pl.ds(s * DEG_SLICE, DEG_SLICE)],
    )


@functools.partial(
    pl.kernel,
    out_type=jax.ShapeDtypeStruct((NC, N, D), jnp.float32),
    mesh=_MESH,
    scratch_types=[
        pltpu.VMEM((RPT, CHUNK), jnp.int32),
        pltpu.VMEM((RPT, CHUNK), jnp.int32),
        pltpu.VMEM((CHUNK, D), jnp.float32),
        pltpu.VMEM((ROW_SLICE // 5, D), jnp.float32),
        pltpu.VMEM_SHARED((N, D), jnp.float32),
        pltpu.SemaphoreType.DMA,
    ],
)
def _scatter_kernel(y_hbm, src_hbm, dst_hbm, out_hbm,
                    src_v, dst_v, rows_v, zero_v, acc_sh, sem):
    c = lax.axis_index("c")
    s = lax.axis_index("s")
    wid = c * NS + s
    base = wid * RPT

    zrows = ROW_SLICE // 5

    def zfill(k, carry):
        for jj in range(D // 16):
            zero_v[k, pl.ds(jj * 16, 16)] = jnp.zeros((16,), jnp.float32)
        return carry

    lax.fori_loop(0, zrows, zfill, 0)
    for r in range(5):
        pltpu.sync_copy(
            zero_v, acc_sh.at[pl.ds(s * ROW_SLICE + r * zrows, zrows)]
        )

    pltpu.sync_copy(src_hbm.at[pl.ds(base, RPT)], src_v)
    pltpu.sync_copy(dst_hbm.at[pl.ds(base, RPT)], dst_v)
    plsc.subcore_barrier()

    def body(j, carry):
        pltpu.async_copy(y_hbm.at[src_v.at[j]], rows_v, sem).wait()
        pltpu.sync_copy(rows_v, acc_sh.at[dst_v.at[j]], add=True)
        return carry

    lax.fori_loop(0, RPT, body, 0)
    plsc.subcore_barrier()
    pltpu.sync_copy(
        acc_sh.at[pl.ds(s * ROW_SLICE, ROW_SLICE)],
        out_hbm.at[c, pl.ds(s * ROW_SLICE, ROW_SLICE)],
    )


BLK = 1000
GRID = N // BLK


def _k1_body(x_ref, w_ref, deg_ref, y_ref):
    dinv = lax.rsqrt(deg_ref[0] + deg_ref[1] + 1.0)[:, None]
    xw = jnp.dot(x_ref[...], w_ref[...], preferred_element_type=jnp.float32)
    y_ref[...] = xw * dinv


_k1 = pl.pallas_call(
    _k1_body,
    grid=(GRID,),
    in_specs=[
        pl.BlockSpec((BLK, D), lambda i: (i, 0)),
        pl.BlockSpec((D, D), lambda i: (0, 0)),
        pl.BlockSpec((2, BLK), lambda i: (0, i)),
    ],
    out_specs=pl.BlockSpec((BLK, D), lambda i: (i, 0)),
    out_shape=jax.ShapeDtypeStruct((N, D), jnp.float32),
)


def _k2_body(acc_ref, y0_ref, deg_ref, b_ref, w_ref, y1_ref):
    dinv = lax.rsqrt(deg_ref[0] + deg_ref[1] + 1.0)[:, None]
    h = jnp.maximum((acc_ref[0] + acc_ref[1] + y0_ref[...]) * dinv + b_ref[...], 0.0)
    y1_ref[...] = jnp.dot(h, w_ref[...], preferred_element_type=jnp.float32) * dinv


_k2 = pl.pallas_call(
    _k2_body,
    grid=(GRID,),
    in_specs=[
        pl.BlockSpec((2, BLK, D), lambda i: (0, i, 0)),
        pl.BlockSpec((BLK, D), lambda i: (i, 0)),
        pl.BlockSpec((2, BLK), lambda i: (0, i)),
        pl.BlockSpec((1, D), lambda i: (0, 0)),
        pl.BlockSpec((D, D), lambda i: (0, 0)),
    ],
    out_specs=pl.BlockSpec((BLK, D), lambda i: (i, 0)),
    out_shape=jax.ShapeDtypeStruct((N, D), jnp.float32),
)


def _k3_body(acc_ref, y1_ref, deg_ref, b_ref, out_ref):
    dinv = lax.rsqrt(deg_ref[0] + deg_ref[1] + 1.0)[:, None]
    out_ref[...] = jnp.maximum(
        (acc_ref[0] + acc_ref[1] + y1_ref[...]) * dinv + b_ref[...], 0.0
    )


_k3 = pl.pallas_call(
    _k3_body,
    grid=(GRID,),
    in_specs=[
        pl.BlockSpec((2, BLK, D), lambda i: (0, i, 0)),
        pl.BlockSpec((BLK, D), lambda i: (i, 0)),
        pl.BlockSpec((2, BLK), lambda i: (0, i)),
        pl.BlockSpec((1, D), lambda i: (0, 0)),
    ],
    out_specs=pl.BlockSpec((BLK, D), lambda i: (i, 0)),
    out_shape=jax.ShapeDtypeStruct((N, D), jnp.float32),
)


def kernel(edge_index, emb, W0, b0, W1, b1):
    src = edge_index[0].astype(jnp.int32).reshape(NROWS, CHUNK)
    dst = edge_index[1].astype(jnp.int32).reshape(NROWS, CHUNK)
    b0r = b0.reshape(1, D)
    b1r = b1.reshape(1, D)

    deg2 = _deg_kernel(dst)
    deg2 = deg2[:, :N]
    y0 = _k1(emb, W0, deg2)
    acc0 = _scatter_kernel(y0, src, dst)
    y1 = _k2(acc0, y0, deg2, b0r, W1)
    acc1 = _scatter_kernel(y1, src, dst)
    return _k3(acc1, y1, deg2, b1r)


# speedup vs baseline: 16.3748x; 16.3748x over previous
import functools

import jax
import jax.numpy as jnp
import numpy as np
from jax import lax
from jax.experimental import pallas as pl
from jax.experimental.pallas import tpu as pltpu
from jax.experimental.pallas import tpu_sc as plsc

N = 10000
D = 128
DH = D // 2
E = 320000

NC = 2
NS = 16
NW = NC * NS
CHUNK = 128
RPT = 80
NROWS = NW * RPT
RPT_SC = NROWS // NS
MPAIR = RPT_SC // 2
E_PAD = NROWS * CHUNK
NP = 10240
ROW_SLICE = NP // NS
ZROWS = ROW_SLICE // 5

_CM = np.empty((D,), np.int32)
for _base in range(0, D, 32):
    for _t in range(16):
        _CM[_base + 2 * _t] = _base + _t
        _CM[_base + 2 * _t + 1] = _base + 16 + _t
_COLMAP = _CM

_MESH = plsc.VectorSubcoreMesh(core_axis_name="c", subcore_axis_name="s")


@functools.partial(
    pl.kernel,
    out_type=jax.ShapeDtypeStruct((NC, NP), jnp.float32),
    mesh=_MESH,
    scratch_types=[
        pltpu.VMEM((RPT, CHUNK), jnp.int32),
        pltpu.VMEM((CHUNK,), jnp.float32),
        pltpu.VMEM((ROW_SLICE,), jnp.float32),
        pltpu.VMEM_SHARED((NP,), jnp.float32),
    ],
)
def _deg_kernel(dst_hbm, out_hbm, dst_v, ones_v, zero_v, acc_sh):
    c = lax.axis_index("c")
    s = lax.axis_index("s")
    wid = c * NS + s

    for j in range(CHUNK // 16):
        ones_v[pl.ds(j * 16, 16)] = jnp.full((16,), 1.0, jnp.float32)

    def zfill(k, carry):
        zero_v[pl.ds(k * 16, 16)] = jnp.zeros((16,), jnp.float32)
        return carry

    lax.fori_loop(0, ROW_SLICE // 16, zfill, 0)
    pltpu.sync_copy(zero_v, acc_sh.at[pl.ds(s * ROW_SLICE, ROW_SLICE)])
    pltpu.sync_copy(dst_hbm.at[pl.ds(wid * RPT, RPT)], dst_v)
    plsc.subcore_barrier()

    def body(j, carry):
        pltpu.sync_copy(ones_v, acc_sh.at[dst_v.at[j]], add=True)
        return carry

    lax.fori_loop(0, RPT, body, 0)
    plsc.subcore_barrier()
    pltpu.sync_copy(
        acc_sh.at[pl.ds(s * ROW_SLICE, ROW_SLICE)],
        out_hbm.at[c, pl.ds(s * ROW_SLICE, ROW_SLICE)],
    )


@functools.partial(
    pl.kernel,
    out_type=jax.ShapeDtypeStruct((NC, NP, DH), jnp.float32),
    mesh=_MESH,
    compiler_params=pltpu.CompilerParams(
        use_tc_tiling_on_sc=False, needs_layout_passes=False,
        internal_scratch_in_bytes=262144),
    scratch_types=[
        pltpu.VMEM((RPT_SC, CHUNK), jnp.int32),
        pltpu.VMEM((RPT_SC, CHUNK), jnp.int32),
        pltpu.VMEM((2, CHUNK, DH), jnp.bfloat16),
        pltpu.VMEM((2, CHUNK, DH), jnp.float32),
        pltpu.VMEM((ZROWS, DH), jnp.float32),
        pltpu.VMEM_SHARED((NP, DH), jnp.float32),
        pltpu.SemaphoreType.DMA,
        pltpu.SemaphoreType.DMA,
        pltpu.SemaphoreType.DMA,
        pltpu.SemaphoreType.DMA,
    ],
)
def _scatter_kernel(ybf_hbm, src_hbm, dst_hbm, out_hbm,
                    src_v, dst_v, rows_bf, rows_f, zero_v, acc_sh,
                    gsem0, gsem1, ssem0, ssem1):
    c = lax.axis_index("c")
    s = lax.axis_index("s")
    base = s * RPT_SC
    row0 = s * ROW_SLICE

    def zfill(k, carry):
        for jj in range(DH // 16):
            zero_v[k, pl.ds(jj * 16, 16)] = jnp.zeros((16,), jnp.float32)
        return carry

    lax.fori_loop(0, ZROWS, zfill, 0)
    for r in range(5):
        pltpu.sync_copy(
            zero_v, acc_sh.at[pl.ds(row0 + r * ZROWS, ZROWS)]
        )
    pltpu.sync_copy(src_hbm.at[c, pl.ds(base, RPT_SC)], src_v)
    pltpu.sync_copy(dst_hbm.at[pl.ds(base, RPT_SC)], dst_v)
    plsc.subcore_barrier()

    gsems = (gsem0, gsem1)
    ssems = (ssem0, ssem1)

    def gather(chunk, buf):
        pltpu.async_copy(
            ybf_hbm.at[src_v.at[chunk]], rows_bf.at[buf], gsems[buf])

    def wait_gather(chunk, buf):
        pltpu.make_async_copy(
            ybf_hbm.at[src_v.at[chunk]], rows_bf.at[buf], gsems[buf]).wait()

    def unpack(buf):
        def urow(r, carry):
            for g in range(DH // 32):
                v = rows_bf[buf, r, pl.ds(g * 32, 32)]
                ev, od = plsc.unpack(v, format=plsc.PackFormat.INTERLEAVED)
                rows_f[buf, r, pl.ds(g * 32, 16)] = ev
                rows_f[buf, r, pl.ds(g * 32 + 16, 16)] = od
            return carry

        lax.fori_loop(0, CHUNK, urow, 0)

    def scatter(chunk, buf):
        pltpu.async_copy(
            rows_f.at[buf], acc_sh.at[dst_v.at[chunk]], ssems[buf], add=True)

    def wait_scatter(chunk, buf):
        pltpu.make_async_copy(
            rows_f.at[buf], acc_sh.at[dst_v.at[chunk]], ssems[buf]).wait()

    gather(0, 0)
    gather(1, 1)
    wait_gather(0, 0)
    unpack(0)
    scatter(0, 0)
    gather(2, 0)
    wait_gather(1, 1)
    unpack(1)
    scatter(1, 1)
    gather(3, 1)

    def body(j, carry):
        a = 2 * j
        wait_scatter(a - 2, 0)
        wait_gather(a, 0)
        unpack(0)
        scatter(a, 0)
        gather(a + 2, 0)
        wait_scatter(a - 1, 1)
        wait_gather(a + 1, 1)
        unpack(1)
        scatter(a + 1, 1)
        gather(a + 3, 1)
        return carry

    lax.fori_loop(1, MPAIR - 1, body, 0)
    a = RPT_SC - 2
    wait_scatter(a - 2, 0)
    wait_gather(a, 0)
    unpack(0)
    scatter(a, 0)
    wait_scatter(a - 1, 1)
    wait_gather(a + 1, 1)
    unpack(1)
    scatter(a + 1, 1)
    wait_scatter(a, 0)
    wait_scatter(a + 1, 1)
    plsc.subcore_barrier()
    pltpu.sync_copy(
        acc_sh.at[pl.ds(row0, ROW_SLICE)],
        out_hbm.at[c, pl.ds(row0, ROW_SLICE)],
    )


BLK = 2000
GRID = N // BLK


def _dinv_col(deg_ref):
    t = deg_ref[...]
    return lax.rsqrt(t[:, 0] + t[:, 1] + 1.0)[:, None]


def _k1_body(x_ref, w_ref, wp_ref, deg_ref, y_ref, ybf_ref):
    dinv = _dinv_col(deg_ref)
    x = x_ref[...]
    y = jnp.dot(x, w_ref[...], preferred_element_type=jnp.float32) * dinv
    y_ref[0] = y[:, :DH]
    y_ref[1] = y[:, DH:]
    yp = jnp.dot(x, wp_ref[...], preferred_element_type=jnp.float32) * dinv
    ypb = yp.astype(jnp.bfloat16)
    ybf_ref[0] = ypb[:, :DH]
    ybf_ref[1] = ypb[:, DH:]


_k1 = pl.pallas_call(
    _k1_body,
    grid=(GRID,),
    in_specs=[
        pl.BlockSpec((BLK, D), lambda i: (i, 0)),
        pl.BlockSpec((D, D), lambda i: (0, 0)),
        pl.BlockSpec((D, D), lambda i: (0, 0)),
        pl.BlockSpec((BLK, 2), lambda i: (i, 0)),
    ],
    out_specs=[
        pl.BlockSpec((2, BLK, DH), lambda i: (0, i, 0)),
        pl.BlockSpec((2, BLK, DH), lambda i: (0, i, 0)),
    ],
    out_shape=[
        jax.ShapeDtypeStruct((2, N, DH), jnp.float32),
        jax.ShapeDtypeStruct((NC, NP, DH), jnp.bfloat16),
    ],
)


def _k2_body(acc_ref, y0_ref, deg_ref, b_ref, w_ref, wp_ref, y1_ref, ybf_ref):
    dinv = _dinv_col(deg_ref)
    t = jnp.concatenate(
        [acc_ref[0] + y0_ref[0], acc_ref[1] + y0_ref[1]], axis=1
    )
    h = jnp.maximum(t * dinv + b_ref[...], 0.0)
    y1 = jnp.dot(h, w_ref[...], preferred_element_type=jnp.float32) * dinv
    y1_ref[0] = y1[:, :DH]
    y1_ref[1] = y1[:, DH:]
    yp = jnp.dot(h, wp_ref[...], preferred_element_type=jnp.float32) * dinv
    ypb = yp.astype(jnp.bfloat16)
    ybf_ref[0] = ypb[:, :DH]
    ybf_ref[1] = ypb[:, DH:]


_k2 = pl.pallas_call(
    _k2_body,
    grid=(GRID,),
    in_specs=[
        pl.BlockSpec((2, BLK, DH), lambda i: (0, i, 0)),
        pl.BlockSpec((2, BLK, DH), lambda i: (0, i, 0)),
        pl.BlockSpec((BLK, 2), lambda i: (i, 0)),
        pl.BlockSpec((1, D), lambda i: (0, 0)),
        pl.BlockSpec((D, D), lambda i: (0, 0)),
        pl.BlockSpec((D, D), lambda i: (0, 0)),
    ],
    out_specs=[
        pl.BlockSpec((2, BLK, DH), lambda i: (0, i, 0)),
        pl.BlockSpec((2, BLK, DH), lambda i: (0, i, 0)),
    ],
    out_shape=[
        jax.ShapeDtypeStruct((2, N, DH), jnp.float32),
        jax.ShapeDtypeStruct((NC, NP, DH), jnp.bfloat16),
    ],
)


def _k3_body(acc_ref, y1_ref, deg_ref, b_ref, out_ref):
    dinv = _dinv_col(deg_ref)
    t = jnp.concatenate(
        [acc_ref[0] + y1_ref[0], acc_ref[1] + y1_ref[1]], axis=1
    )
    out_ref[...] = jnp.maximum(t * dinv + b_ref[...], 0.0)


_k3 = pl.pallas_call(
    _k3_body,
    grid=(GRID,),
    in_specs=[
        pl.BlockSpec((2, BLK, DH), lambda i: (0, i, 0)),
        pl.BlockSpec((2, BLK, DH), lambda i: (0, i, 0)),
        pl.BlockSpec((BLK, 2), lambda i: (i, 0)),
        pl.BlockSpec((1, D), lambda i: (0, 0)),
    ],
    out_specs=pl.BlockSpec((BLK, D), lambda i: (i, 0)),
    out_shape=jax.ShapeDtypeStruct((N, D), jnp.float32),
)


def kernel(edge_index, emb, W0, b0, W1, b1):
    pad = E_PAD - E
    src1 = jnp.concatenate(
        [edge_index[0].astype(jnp.int32), jnp.zeros((pad,), jnp.int32)]
    )
    src = jnp.stack([src1, src1 + NP]).reshape(NC, NROWS, CHUNK)
    dst = jnp.concatenate(
        [edge_index[1].astype(jnp.int32), jnp.full((pad,), N, jnp.int32)]
    ).reshape(NROWS, CHUNK)
    b0r = b0.reshape(1, D)
    b1r = b1.reshape(1, D)
    W0p = W0[:, _COLMAP]
    W1p = W1[:, _COLMAP]

    deg2 = _deg_kernel(dst)
    degT = deg2[:, :N].T
    y0, ybf0 = _k1(emb, W0, W0p, degT)
    acc0 = _scatter_kernel(ybf0.reshape(NC * NP, DH), src, dst)
    y1, ybf1 = _k2(acc0, y0, degT, b0r, W1, W1p)
    acc1 = _scatter_kernel(ybf1.reshape(NC * NP, DH), src, dst)
    return _k3(acc1, y1, degT, b1r)
